# baseline (speedup 1.0000x reference)
import jax
import jax.numpy as jnp
from jax import lax
from jax.experimental import pallas as pl
from jax.experimental.pallas import tpu as pltpu

_ROWS = 4096
_COLS = 512
_BLK = 1024
_NEG_RATIO = 3.0
_EPS = 1e-6
_LOG2E = 1.4426950408889634
_LN2 = 0.6931471805599453


def _loss(x, t):
    return jnp.maximum(x, 0.0) - x * t + jnp.log1p(jnp.exp(-jnp.abs(x)))


def _block_spec():
    return pl.BlockSpec((_BLK, _COLS), lambda i: (i, 0))


def _stats_body(x_ref, t_ref, m_ref, out_ref, acc_ref):
    i = pl.program_id(0)

    @pl.when(i == 0)
    def _():
        acc_ref[...] = jnp.zeros_like(acc_ref)

    x = x_ref[...]
    t = t_ref[...]
    m = m_ref[...]
    e = jnp.exp2(jnp.abs(x) * (-_LOG2E))
    q = jnp.maximum(x, 0.0) + _LN2 * jnp.log2(1.0 + e)
    tm = t * m

    ones = jnp.ones((1, _BLK), jnp.float32)

    def colsum(v):
        return lax.dot_general(
            ones, v, (((1,), (0,)), ((), ())),
            preferred_element_type=jnp.float32,
        )

    acc_ref[0:1, :] += colsum(m * q)
    acc_ref[1:2, :] += colsum(tm * x)
    acc_ref[2:3, :] += colsum(tm * q)
    acc_ref[3:4, :] += colsum(tm)
    acc_ref[4:5, :] += colsum(m)

    @pl.when(i == pl.num_programs(0) - 1)
    def _():
        s_mq = jnp.sum(acc_ref[0, :])
        s_tmx = jnp.sum(acc_ref[1, :])
        s_tmq = jnp.sum(acc_ref[2, :])
        s_tm = jnp.sum(acc_ref[3, :])
        s_m = jnp.sum(acc_ref[4, :])
        out_ref[0] = s_tmq - s_tmx
        out_ref[1] = s_mq - s_tmq
        out_ref[2] = s_tm
        out_ref[3] = s_m - s_tm


def _run_stats(x, t, m):
    spec = _block_spec()
    return pl.pallas_call(
        _stats_body,
        grid=(_ROWS // _BLK,),
        in_specs=[spec, spec, spec],
        out_specs=pl.BlockSpec(memory_space=pltpu.SMEM),
        out_shape=jax.ShapeDtypeStruct((4,), jnp.float32),
        scratch_shapes=[pltpu.VMEM((8, _COLS), jnp.float32)],
    )(x, t, m)


def _count_body(thr_ref, x_ref, t_ref, m_ref, out_ref):
    @pl.when(pl.program_id(0) == 0)
    def _():
        out_ref[0] = 0.0
        out_ref[1] = 0.0

    x = x_ref[...]
    neg = m_ref[...] * (1.0 - t_ref[...])
    b = lax.bitcast_convert_type(x, jnp.int32)
    skey = jnp.where(b >= 0, b, b ^ jnp.int32(0x7FFFFFFF))
    sel = (skey > thr_ref[0]) & (neg > 0.0)
    out_ref[0] += jnp.sum(jnp.where(sel, 1.0, 0.0))
    out_ref[1] += jnp.sum(jnp.where(sel, _loss(x, 0.0), 0.0))


def _count_sum_above(x, t, m, thr_s):
    spec = _block_spec()
    out = pl.pallas_call(
        _count_body,
        grid=(_ROWS // _BLK,),
        in_specs=[pl.BlockSpec(memory_space=pltpu.SMEM), spec, spec, spec],
        out_specs=pl.BlockSpec(memory_space=pltpu.SMEM),
        out_shape=jax.ShapeDtypeStruct((2,), jnp.float32),
    )(jnp.reshape(thr_s, (1,)).astype(jnp.int32), x, t, m)
    return out[0], out[1]


def _signed_thr(biased_u):
    return lax.bitcast_convert_type(biased_u ^ jnp.uint32(0x80000000), jnp.int32)


def _topk_slow(x, t, m, k):
    def body(_, lohi):
        lo, hi = lohi
        mid = lo + (hi - lo) // jnp.uint32(2)
        cnt, _s = _count_sum_above(x, t, m, _signed_thr(mid))
        pred = cnt < k
        lo2 = jnp.where(pred, lo, mid + jnp.uint32(1))
        hi2 = jnp.where(pred, mid, hi)
        return lo2, hi2

    lo0 = jnp.uint32(0)
    hi0 = jnp.uint32(0xFFFFFFFF)
    _lo, tau = lax.fori_loop(0, 32, body, (lo0, hi0))
    cnt_a, sum_a = _count_sum_above(x, t, m, _signed_thr(tau))
    u_bits = jnp.where(
        tau >= jnp.uint32(0x80000000), tau ^ jnp.uint32(0x80000000), ~tau
    )
    x_tau = lax.bitcast_convert_type(u_bits, jnp.float32)
    tie_loss = jnp.maximum(x_tau, 0.0) + jnp.log1p(jnp.exp(-jnp.abs(x_tau)))
    partial = jnp.where(k > cnt_a, (k - cnt_a) * tie_loss, 0.0)
    return sum_a + partial


def kernel(output, target, mask):
    x = output.reshape(_ROWS, _COLS)
    t = target.reshape(_ROWS, _COLS)
    m = mask.reshape(_ROWS, _COLS)
    s = _run_stats(x, t, m)
    pos_loss, neg_loss, pos_cnt, neg_cnt = s[0], s[1], s[2], s[3]
    k = jnp.minimum(neg_cnt, _NEG_RATIO * pos_cnt)
    topk = neg_loss
    return (pos_loss + topk) / (pos_cnt + k + _EPS)


# speedup vs baseline: 1.0152x; 1.0152x over previous
import jax
import jax.numpy as jnp
from jax import lax
from jax.experimental import pallas as pl
from jax.experimental.pallas import tpu as pltpu

_ROWS = 4096
_COLS = 512
_BLK = 1024
_NEG_RATIO = 3.0
_EPS = 1e-6
_LOG2E = 1.4426950408889634
_LN2 = 0.6931471805599453


def _loss(x, t):
    return jnp.maximum(x, 0.0) - x * t + jnp.log1p(jnp.exp(-jnp.abs(x)))


def _block_spec():
    return pl.BlockSpec((_BLK, _COLS), lambda i: (i, 0))


def _stats_body(x_ref, t_ref, m_ref, out_ref, acc_ref):
    i = pl.program_id(0)

    @pl.when(i == 0)
    def _():
        acc_ref[...] = jnp.zeros_like(acc_ref)

    x = x_ref[...]
    t = t_ref[...]
    m = m_ref[...]
    e = jnp.exp2(jnp.abs(x) * (-_LOG2E))
    q = jnp.maximum(x, 0.0) + _LN2 * jnp.log2(1.0 + e)
    tm = t * m

    ones = jnp.ones((1, _BLK), jnp.float32)

    def colsum(v):
        return lax.dot_general(
            ones, v, (((1,), (0,)), ((), ())),
            preferred_element_type=jnp.float32,
        )

    acc_ref[0:1, :] += colsum(m * q)
    acc_ref[1:2, :] += colsum(tm * x)
    acc_ref[2:3, :] += colsum(tm * q)
    acc_ref[3:4, :] += colsum(tm)
    acc_ref[4:5, :] += colsum(m)

    @pl.when(i == pl.num_programs(0) - 1)
    def _():
        s_mq = jnp.sum(acc_ref[0, :])
        s_tmx = jnp.sum(acc_ref[1, :])
        s_tmq = jnp.sum(acc_ref[2, :])
        s_tm = jnp.sum(acc_ref[3, :])
        s_m = jnp.sum(acc_ref[4, :])
        pos_loss = s_tmq - s_tmx
        neg_loss = s_mq - s_tmq
        pos_cnt = s_tm
        neg_cnt = s_m - s_tm
        k = jnp.minimum(neg_cnt, _NEG_RATIO * pos_cnt)
        out_ref[0] = (pos_loss + neg_loss) / (pos_cnt + k + _EPS)
        out_ref[1] = neg_cnt - _NEG_RATIO * pos_cnt
        out_ref[2] = pos_loss
        out_ref[3] = pos_cnt


def _run_stats(x, t, m):
    spec = _block_spec()
    return pl.pallas_call(
        _stats_body,
        grid=(_ROWS // _BLK,),
        in_specs=[spec, spec, spec],
        out_specs=pl.BlockSpec(memory_space=pltpu.SMEM),
        out_shape=jax.ShapeDtypeStruct((4,), jnp.float32),
        scratch_shapes=[pltpu.VMEM((8, _COLS), jnp.float32)],
    )(x, t, m)


def _count_body(thr_ref, x_ref, t_ref, m_ref, out_ref):
    @pl.when(pl.program_id(0) == 0)
    def _():
        out_ref[0] = 0.0
        out_ref[1] = 0.0

    x = x_ref[...]
    neg = m_ref[...] * (1.0 - t_ref[...])
    b = lax.bitcast_convert_type(x, jnp.int32)
    skey = jnp.where(b >= 0, b, b ^ jnp.int32(0x7FFFFFFF))
    sel = (skey > thr_ref[0]) & (neg > 0.0)
    out_ref[0] += jnp.sum(jnp.where(sel, 1.0, 0.0))
    out_ref[1] += jnp.sum(jnp.where(sel, _loss(x, 0.0), 0.0))


def _count_sum_above(x, t, m, thr_s):
    spec = _block_spec()
    out = pl.pallas_call(
        _count_body,
        grid=(_ROWS // _BLK,),
        in_specs=[pl.BlockSpec(memory_space=pltpu.SMEM), spec, spec, spec],
        out_specs=pl.BlockSpec(memory_space=pltpu.SMEM),
        out_shape=jax.ShapeDtypeStruct((2,), jnp.float32),
    )(jnp.reshape(thr_s, (1,)).astype(jnp.int32), x, t, m)
    return out[0], out[1]


def _signed_thr(biased_u):
    return lax.bitcast_convert_type(biased_u ^ jnp.uint32(0x80000000), jnp.int32)


def _topk_slow(x, t, m, k):
    def body(_, lohi):
        lo, hi = lohi
        mid = lo + (hi - lo) // jnp.uint32(2)
        cnt, _s = _count_sum_above(x, t, m, _signed_thr(mid))
        pred = cnt < k
        lo2 = jnp.where(pred, lo, mid + jnp.uint32(1))
        hi2 = jnp.where(pred, mid, hi)
        return lo2, hi2

    lo0 = jnp.uint32(0)
    hi0 = jnp.uint32(0xFFFFFFFF)
    _lo, tau = lax.fori_loop(0, 32, body, (lo0, hi0))
    cnt_a, sum_a = _count_sum_above(x, t, m, _signed_thr(tau))
    u_bits = jnp.where(
        tau >= jnp.uint32(0x80000000), tau ^ jnp.uint32(0x80000000), ~tau
    )
    x_tau = lax.bitcast_convert_type(u_bits, jnp.float32)
    tie_loss = jnp.maximum(x_tau, 0.0) + jnp.log1p(jnp.exp(-jnp.abs(x_tau)))
    partial = jnp.where(k > cnt_a, (k - cnt_a) * tie_loss, 0.0)
    return sum_a + partial


def kernel(output, target, mask):
    x = output.reshape(_ROWS, _COLS)
    t = target.reshape(_ROWS, _COLS)
    m = mask.reshape(_ROWS, _COLS)
    s = _run_stats(x, t, m)

    def _slow():
        pos_loss, pos_cnt = s[2], s[3]
        k = _NEG_RATIO * pos_cnt
        topk = _topk_slow(x, t, m, k)
        return (pos_loss + topk) / (pos_cnt + k + _EPS)

    return lax.cond(s[1] <= 0.0, lambda: s[0], _slow)


# speedup vs baseline: 1.5488x; 1.5256x over previous
import jax
import jax.numpy as jnp
from jax import lax
from jax.experimental import pallas as pl
from jax.experimental.pallas import tpu as pltpu

_ROWS = 4096
_COLS = 512
_BROWS = 512
_NBLK = _ROWS // _BROWS
_NEG_RATIO = 3.0
_EPS = 1e-6
_LOG2E = 1.4426950408889634
_LN2 = 0.6931471805599453


def _softplus(x):
    e = jnp.exp2(jnp.abs(x) * (-_LOG2E))
    return jnp.maximum(x, 0.0) + _LN2 * jnp.log2(1.0 + e)


def _fused_body(x_hbm, t_hbm, m_hbm, out_ref, x_v, t_v, m_v, sems):
    copies = []
    for b in range(_NBLK):
        rows = pl.ds(b * _BROWS, _BROWS)
        for j, (src, dst) in enumerate(
            ((x_hbm, x_v), (t_hbm, t_v), (m_hbm, m_v))
        ):
            cp = pltpu.make_async_copy(
                src.at[rows, :], dst.at[rows, :], sems.at[b, j]
            )
            cp.start()
            copies.append(cp)

    ones = jnp.ones((1, _BROWS), jnp.float32)

    def colsum(v):
        return lax.dot_general(
            ones, v, (((1,), (0,)), ((), ())),
            preferred_element_type=jnp.float32,
        )

    z = jnp.zeros((1, _COLS), jnp.float32)
    s_mq, s_tmx, s_tmq, s_tm, s_m = z, z, z, z, z
    for b in range(_NBLK):
        for cp in copies[3 * b:3 * b + 3]:
            cp.wait()
        rows = pl.ds(b * _BROWS, _BROWS)
        x = x_v[rows, :]
        t = t_v[rows, :]
        m = m_v[rows, :]
        q = _softplus(x)
        tm = t * m
        s_mq = s_mq + colsum(m * q)
        s_tmx = s_tmx + colsum(tm * x)
        s_tmq = s_tmq + colsum(tm * q)
        s_tm = s_tm + colsum(tm)
        s_m = s_m + colsum(m)

    pos_loss = jnp.sum(s_tmq) - jnp.sum(s_tmx)
    neg_loss = jnp.sum(s_mq) - jnp.sum(s_tmq)
    pos_cnt = jnp.sum(s_tm)
    neg_cnt = jnp.sum(s_m) - pos_cnt
    k = jnp.minimum(neg_cnt, _NEG_RATIO * pos_cnt)
    denom = pos_cnt + k + _EPS

    out_ref[0] = (pos_loss + neg_loss) / denom

    @pl.when(neg_cnt > _NEG_RATIO * pos_cnt)
    def _():
        def count_sum_above(thr_i32):
            cnt = jnp.float32(0.0)
            sm = jnp.float32(0.0)
            for b in range(_NBLK):
                rows = pl.ds(b * _BROWS, _BROWS)
                x = x_v[rows, :]
                neg = m_v[rows, :] * (1.0 - t_v[rows, :])
                bkey = lax.bitcast_convert_type(x, jnp.int32)
                skey = jnp.where(bkey >= 0, bkey, bkey ^ jnp.int32(0x7FFFFFFF))
                sel = (skey > thr_i32) & (neg > 0.0)
                cnt = cnt + jnp.sum(jnp.where(sel, 1.0, 0.0))
                sm = sm + jnp.sum(jnp.where(sel, _softplus(x), 0.0))
            return cnt, sm

        def signed(biased_u):
            return lax.bitcast_convert_type(
                biased_u ^ jnp.uint32(0x80000000), jnp.int32
            )

        def body(_, lohi):
            lo, hi = lohi
            mid = lo + ((hi - lo) // jnp.uint32(2))
            cnt, _s = count_sum_above(signed(mid))
            pred = cnt < k
            lo2 = jnp.where(pred, lo, mid + jnp.uint32(1))
            hi2 = jnp.where(pred, mid, hi)
            return lo2, hi2

        _lo, tau = lax.fori_loop(
            0, 32, body, (jnp.uint32(0), jnp.uint32(0xFFFFFFFF))
        )
        cnt_a, sum_a = count_sum_above(signed(tau))
        u_bits = jnp.where(
            tau >= jnp.uint32(0x80000000), tau ^ jnp.uint32(0x80000000), ~tau
        )
        x_tau = lax.bitcast_convert_type(u_bits, jnp.float32)
        tie_loss = _softplus(x_tau)
        partial = jnp.where(k > cnt_a, (k - cnt_a) * tie_loss, 0.0)
        out_ref[0] = (pos_loss + sum_a + partial) / denom


def kernel(output, target, mask):
    x = output.reshape(_ROWS, _COLS)
    t = target.reshape(_ROWS, _COLS)
    m = mask.reshape(_ROWS, _COLS)
    res = pl.pallas_call(
        _fused_body,
        in_specs=[
            pl.BlockSpec(memory_space=pl.ANY),
            pl.BlockSpec(memory_space=pl.ANY),
            pl.BlockSpec(memory_space=pl.ANY),
        ],
        out_specs=pl.BlockSpec(memory_space=pltpu.SMEM),
        out_shape=jax.ShapeDtypeStruct((1,), jnp.float32),
        scratch_shapes=[
            pltpu.VMEM((_ROWS, _COLS), jnp.float32),
            pltpu.VMEM((_ROWS, _COLS), jnp.float32),
            pltpu.VMEM((_ROWS, _COLS), jnp.float32),
            pltpu.SemaphoreType.DMA((_NBLK, 3)),
        ],
    )(x, t, m)
    return res[0]


# speedup vs baseline: 1.5614x; 1.0081x over previous
import jax
import jax.numpy as jnp
from jax import lax
from jax.experimental import pallas as pl
from jax.experimental.pallas import tpu as pltpu

_ROWS = 4096
_COLS = 512
_BROWS = 256
_NBLK = _ROWS // _BROWS
_NEG_RATIO = 3.0
_EPS = 1e-6
_LOG2E = 1.4426950408889634
_LN2 = 0.6931471805599453


def _softplus(x):
    e = jnp.exp2(jnp.abs(x) * (-_LOG2E))
    return jnp.maximum(x, 0.0) + _LN2 * jnp.log2(1.0 + e)


def _fused_body(x_hbm, t_hbm, m_hbm, out_ref, x_v, t_v, m_v, sems):
    copies = []
    for b in range(_NBLK):
        rows = pl.ds(b * _BROWS, _BROWS)
        for j, (src, dst) in enumerate(
            ((x_hbm, x_v), (t_hbm, t_v), (m_hbm, m_v))
        ):
            cp = pltpu.make_async_copy(
                src.at[rows, :], dst.at[rows, :], sems.at[b, j]
            )
            cp.start()
            copies.append(cp)

    ones = jnp.ones((1, _BROWS), jnp.float32)

    def colsum(v):
        return lax.dot_general(
            ones, v, (((1,), (0,)), ((), ())),
            preferred_element_type=jnp.float32,
        )

    z = jnp.zeros((1, _COLS), jnp.float32)
    s_mq, s_tmx, s_tmq, s_tm, s_m = z, z, z, z, z
    for b in range(_NBLK):
        for cp in copies[3 * b:3 * b + 3]:
            cp.wait()
        rows = pl.ds(b * _BROWS, _BROWS)
        x = x_v[rows, :]
        t = t_v[rows, :]
        m = m_v[rows, :]
        q = _softplus(x)
        tm = t * m
        s_mq = s_mq + colsum(m * q)
        s_tmx = s_tmx + colsum(tm * x)
        s_tmq = s_tmq + colsum(tm * q)
        s_tm = s_tm + colsum(tm)
        s_m = s_m + colsum(m)

    pos_loss = jnp.sum(s_tmq) - jnp.sum(s_tmx)
    neg_loss = jnp.sum(s_mq) - jnp.sum(s_tmq)
    pos_cnt = jnp.sum(s_tm)
    neg_cnt = jnp.sum(s_m) - pos_cnt
    k = jnp.minimum(neg_cnt, _NEG_RATIO * pos_cnt)
    denom = pos_cnt + k + _EPS

    out_ref[0] = (pos_loss + neg_loss) / denom

    @pl.when(neg_cnt > _NEG_RATIO * pos_cnt)
    def _():
        def count_sum_above(thr_i32):
            cnt = jnp.float32(0.0)
            sm = jnp.float32(0.0)
            for b in range(_NBLK):
                rows = pl.ds(b * _BROWS, _BROWS)
                x = x_v[rows, :]
                neg = m_v[rows, :] * (1.0 - t_v[rows, :])
                bkey = lax.bitcast_convert_type(x, jnp.int32)
                skey = jnp.where(bkey >= 0, bkey, bkey ^ jnp.int32(0x7FFFFFFF))
                sel = (skey > thr_i32) & (neg > 0.0)
                cnt = cnt + jnp.sum(jnp.where(sel, 1.0, 0.0))
                sm = sm + jnp.sum(jnp.where(sel, _softplus(x), 0.0))
            return cnt, sm

        def signed(biased_u):
            return lax.bitcast_convert_type(
                biased_u ^ jnp.uint32(0x80000000), jnp.int32
            )

        def body(_, lohi):
            lo, hi = lohi
            mid = lo + ((hi - lo) // jnp.uint32(2))
            cnt, _s = count_sum_above(signed(mid))
            pred = cnt < k
            lo2 = jnp.where(pred, lo, mid + jnp.uint32(1))
            hi2 = jnp.where(pred, mid, hi)
            return lo2, hi2

        _lo, tau = lax.fori_loop(
            0, 32, body, (jnp.uint32(0), jnp.uint32(0xFFFFFFFF))
        )
        cnt_a, sum_a = count_sum_above(signed(tau))
        u_bits = jnp.where(
            tau >= jnp.uint32(0x80000000), tau ^ jnp.uint32(0x80000000), ~tau
        )
        x_tau = lax.bitcast_convert_type(u_bits, jnp.float32)
        tie_loss = _softplus(x_tau)
        partial = jnp.where(k > cnt_a, (k - cnt_a) * tie_loss, 0.0)
        out_ref[0] = (pos_loss + sum_a + partial) / denom


def kernel(output, target, mask):
    x = output.reshape(_ROWS, _COLS)
    t = target.reshape(_ROWS, _COLS)
    m = mask.reshape(_ROWS, _COLS)
    res = pl.pallas_call(
        _fused_body,
        in_specs=[
            pl.BlockSpec(memory_space=pl.ANY),
            pl.BlockSpec(memory_space=pl.ANY),
            pl.BlockSpec(memory_space=pl.ANY),
        ],
        out_specs=pl.BlockSpec(memory_space=pltpu.SMEM),
        out_shape=jax.ShapeDtypeStruct((1,), jnp.float32),
        scratch_shapes=[
            pltpu.VMEM((_ROWS, _COLS), jnp.float32),
            pltpu.VMEM((_ROWS, _COLS), jnp.float32),
            pltpu.VMEM((_ROWS, _COLS), jnp.float32),
            pltpu.SemaphoreType.DMA((_NBLK, 3)),
        ],
    )(x, t, m)
    return res[0]
